# baseline (speedup 1.0000x reference)
import functools

import jax
import jax.numpy as jnp
from jax import lax
from jax.experimental import pallas as pl
from jax.experimental.pallas import tpu as pltpu
from jax.experimental.pallas import tpu_sc as plsc

B = 4096
L = 50
E = 64
NUM_SINGLE = 4
NUM_MULTI = 3
N_DENSE = 8
FDIM = 840
FPAD = 896
EPS = 1e-5

NC = 2
NS = 16
NW = NC * NS
BPW = B // NW
CB = 8
NCHUNK = BPW // CB
NSUB = 5
SUBI = CB * L // NSUB
IDXROWS = BPW * L // SUBI


VROWS = 100000
VPAD = 100096
NCHT = VPAD // 128
CPWT = 25


def _sc_relayout(tu_h, tf_h, ta_h, ou_h, of_h, oa_h, bin_v, bout0_v,
                 bout1_v, sem_i, sem_o):
    wid = lax.axis_index("s") * NC + lax.axis_index("c")
    iota16 = lax.iota(jnp.int32, 16)
    bouts = (bout0_v, bout1_v)

    def valid(c):
        return jnp.logical_and(c < CPWT, wid * CPWT + c < NCHT)

    for t_h, o_h in ((tu_h, ou_h), (tf_h, of_h), (ta_h, oa_h)):
        def fire(c, k, t_h=t_h):
            ch = wid * CPWT + c

            @pl.when(valid(c))
            def _():
                pltpu.async_copy(t_h.at[:, pl.ds(ch * 128, 128)],
                                 bin_v.at[k, :, pl.ds(0, 128)], sem_i)

        fire(0, 0)
        fire(1, 1)

        def body(j, _, t_h=t_h, o_h=o_h, fire=fire):
            for k in range(2):
                c = j * 2 + k
                ch = wid * CPWT + c

                @pl.when(valid(c))
                def _():
                    pltpu.make_async_copy(t_h.at[:, pl.ds(0, 128)],
                                          bin_v.at[k, :, pl.ds(0, 128)],
                                          sem_i).wait()
                    def trow(r, _, k=k):
                        rv = jnp.full((16,), r, jnp.int32)
                        for eg in range(4):
                            v = plsc.load_gather(
                                bin_v.at[k], [iota16 + 16 * eg, rv])
                            bouts[k][pl.ds(r * E + 16 * eg, 16)] = v
                        return 0

                    lax.fori_loop(0, 128, trow, 0, unroll=8)
                    fire(c + 2, k)
                    pltpu.sync_copy(bouts[k],
                                    o_h.at[pl.ds(ch * 128 * E, 128 * E)])
            return 0

        lax.fori_loop(0, (CPWT + 1) // 2, body, 0)


_TBL = jax.ShapeDtypeStruct((VPAD * E,), jnp.float32)
_relayout_call = functools.partial(
    pl.kernel,
    _sc_relayout,
    out_type=(_TBL, _TBL, _TBL),
    scratch_types=[
        pltpu.VMEM((2, E, 129), jnp.float32),
        pltpu.VMEM((128 * E,), jnp.float32),
        pltpu.VMEM((128 * E,), jnp.float32),
        pltpu.SemaphoreType.DMA,
        pltpu.SemaphoreType.DMA,
    ],
    compiler_params=pltpu.CompilerParams(use_tc_tiling_on_sc=True,
                                         needs_layout_passes=False),
)


def _sc_gather(t_user, t_feed, t_auth, t_dev, sid_h, mid_h, dense_h,
               merged_h, sidx_v, midx_v, srows_v, rows_v, out_v, dtmp_v,
               sem_s, sem_g0, sem_g1):
    wid = lax.axis_index("s") * NC + lax.axis_index("c")
    base = wid * BPW
    tables = (t_user, t_feed, t_auth, t_dev)
    gsems = (sem_g0, sem_g1)

    pltpu.sync_copy(sid_h.at[:, pl.ds(base, BPW)], sidx_v)
    for t in range(4):
        pltpu.async_copy(tables[t].at[sidx_v.at[t]], srows_v.at[t], sem_s)
    for t in range(4):
        pltpu.make_async_copy(tables[t].at[pl.ds(0, BPW)], srows_v.at[t],
                              sem_s).wait()
    for t in range(4):
        pltpu.sync_copy(srows_v.at[t],
                        merged_h.at[pl.ds(base, BPW), pl.ds(E * t, E)])

    pltpu.sync_copy(dense_h.at[pl.ds(base, BPW)], dtmp_v)
    pltpu.sync_copy(dtmp_v, merged_h.at[pl.ds(base, BPW), pl.ds(832, E)])

    pltpu.sync_copy(mid_h.at[:, wid], midx_v)

    mtables = (t_feed, t_auth, t_user)
    inter_t = (1, 2, 0)

    for i in range(NUM_MULTI):
        mtab = mtables[i]
        ti = inter_t[i]
        colbase = 4 * E + 3 * E * i

        def fire(c, k, mtab=mtab, i=i):
            for p in range(NSUB):
                pltpu.async_copy(
                    mtab.at[midx_v.at[i, c * NSUB + p]],
                    rows_v.at[k, pl.ds(p * SUBI, SUBI)],
                    gsems[k])

        def drain(k, mtab=mtab):
            pltpu.make_async_copy(mtab.at[pl.ds(0, CB * L)], rows_v.at[k],
                                  gsems[k]).wait()

        def compute_chunk(k, ti=ti):
            rows_k = rows_v.at[k]

            def b_body(b, _):
                r0 = b * L

                def l_body(l, acc):
                    r = r0 + l
                    a = list(acc)
                    for e in range(4):
                        v = rows_k[r, pl.ds(e * 16, 16)]
                        a[e] = jnp.maximum(a[e], v)
                        a[4 + e] = jnp.minimum(a[4 + e], v)
                    return tuple(a)

                v0s = tuple(rows_k[r0, pl.ds(e * 16, 16)] for e in range(4))
                acc = lax.fori_loop(1, L, l_body, v0s + v0s, unroll=7)
                for e in range(4):
                    sl = pl.ds(e * 16, 16)
                    mx = acc[e]
                    mn = acc[4 + e]
                    s = srows_v[ti, b, sl]
                    out_v[k, b, sl] = mx
                    out_v[k, b, pl.ds(E + e * 16, 16)] = s - mn
                    out_v[k, b, pl.ds(2 * E + e * 16, 16)] = jnp.where(
                        s >= 0.0, s * mx, s * mn)
                return 0

            lax.fori_loop(0, CB, b_body, 0)

        fire(0, 0)
        fire(1, 1)

        def pair_body(j, _, colbase=colbase, fire=fire, drain=drain,
                      compute_chunk=compute_chunk):
            for k in range(2):
                c = j * 2 + k
                drain(k)
                compute_chunk(k)

                @pl.when(c + 2 < NCHUNK)
                def _f():
                    fire(c + 2, k)

                pltpu.sync_copy(
                    out_v.at[k],
                    merged_h.at[pl.ds(base + c * CB, CB),
                                pl.ds(colbase, 3 * E)])
            return 0

        lax.fori_loop(0, NCHUNK // 2, pair_body, 0)


_gather_call = functools.partial(
    pl.kernel,
    _sc_gather,
    out_type=jax.ShapeDtypeStruct((B, FPAD), jnp.float32),
    scratch_types=[
        pltpu.VMEM((4, BPW), jnp.int32),
        pltpu.VMEM((NUM_MULTI, IDXROWS, SUBI), jnp.int32),
        pltpu.VMEM((4, BPW, E), jnp.float32),
        pltpu.VMEM((2, CB * L, E), jnp.float32),
        pltpu.VMEM((2, CB, 3 * E), jnp.float32),
        pltpu.VMEM((BPW, E), jnp.float32),
        pltpu.SemaphoreType.DMA,
        pltpu.SemaphoreType.DMA,
        pltpu.SemaphoreType.DMA,
    ],
    compiler_params=pltpu.CompilerParams(use_tc_tiling_on_sc=False),
)


def _dense_body(x_ref, k0_ref, b0_ref, k1_ref, b1_ref, w1_ref, c1_ref,
                w2_ref, c2_ref, w3_ref, b3_ref, out_ref):
    x = x_ref[...]
    xw = jnp.sum(x * k0_ref[...], axis=1, keepdims=True)
    x1 = x * xw + b0_ref[...] + x
    xw1 = jnp.sum(x1 * k1_ref[...], axis=1, keepdims=True)
    x2 = x * xw1 + b1_ref[...] + x1
    h1 = jnp.dot(x2, w1_ref[...], preferred_element_type=jnp.float32)
    h1 = jnp.maximum(h1 + c1_ref[...], 0.0)
    outs = []
    for c in range(4):
        hc = h1[:, 256 * c:256 * (c + 1)]
        h2 = jnp.dot(hc, w2_ref[c], preferred_element_type=jnp.float32)
        h2 = jnp.maximum(h2 + c2_ref[c], 0.0)
        outs.append(jnp.sum(h2 * w3_ref[c], axis=1, keepdims=True))
    o = jnp.concatenate(outs, axis=1) + b3_ref[...]
    out_ref[...] = jax.nn.sigmoid(o)


TB = 512


def _dense_call(merged, k0p, b0p, k1p, b1p, w1, c1, w2, c2, w3, b3):
    return pl.pallas_call(
        _dense_body,
        grid=(B // TB,),
        in_specs=[
            pl.BlockSpec((TB, FPAD), lambda i: (i, 0)),
            pl.BlockSpec((1, FPAD), lambda i: (0, 0)),
            pl.BlockSpec((1, FPAD), lambda i: (0, 0)),
            pl.BlockSpec((1, FPAD), lambda i: (0, 0)),
            pl.BlockSpec((1, FPAD), lambda i: (0, 0)),
            pl.BlockSpec((FPAD, 1024), lambda i: (0, 0)),
            pl.BlockSpec((1, 1024), lambda i: (0, 0)),
            pl.BlockSpec((4, 256, 128), lambda i: (0, 0, 0)),
            pl.BlockSpec((4, 1, 128), lambda i: (0, 0, 0)),
            pl.BlockSpec((4, 1, 128), lambda i: (0, 0, 0)),
            pl.BlockSpec((1, 4), lambda i: (0, 0)),
        ],
        out_specs=pl.BlockSpec((TB, 4), lambda i: (i, 0)),
        out_shape=jax.ShapeDtypeStruct((B, 4), jnp.float32),
    )(merged, k0p, b0p, k1p, b1p, w1, c1, w2, c2, w3, b3)


def kernel(dense_features, single_id_concat, multi_id_concat, mask_concat,
           params):
    del mask_concat

    sid = single_id_concat[:, 0, :].T.astype(jnp.int32)
    mid = (multi_id_concat.astype(jnp.int32)
           .transpose(2, 0, 1)
           .reshape(NUM_MULTI, NW, IDXROWS, SUBI))
    dense_pad = jnp.concatenate(
        [dense_features.astype(jnp.float32),
         jnp.zeros((B, E - N_DENSE), jnp.float32)], axis=1)

    inv = (1.0 + EPS) ** -0.5
    w1l, c1l, w2l, c2l, w3l = [], [], [], [], []
    for c in range(4):
        g1 = params['h%d_g1' % c] * inv
        w1 = params['h%d_W1' % c] * g1[None, :]
        c1 = params['h%d_b1' % c] * g1 + params['h%d_be1' % c]
        w1l.append(jnp.zeros((FPAD, 256), jnp.float32)
                   .at[:FDIM, :200].set(w1))
        c1l.append(jnp.zeros((256,), jnp.float32).at[:200].set(c1))
        g2 = params['h%d_g2' % c] * inv
        w2 = params['h%d_W2' % c] * g2[None, :]
        c2 = params['h%d_b2' % c] * g2 + params['h%d_be2' % c]
        w2l.append(jnp.zeros((256, 128), jnp.float32)
                   .at[:200, :80].set(w2))
        c2l.append(jnp.zeros((128,), jnp.float32).at[:80].set(c2))
        w3l.append(jnp.zeros((128,), jnp.float32)
                   .at[:80].set(params['h%d_W3' % c][:, 0]))
    w1cat = jnp.concatenate(w1l, axis=1)
    c1cat = jnp.concatenate(c1l)[None]
    w2s = jnp.stack(w2l)
    c2s = jnp.stack(c2l)[:, None, :]
    w3s = jnp.stack(w3l)[:, None, :]
    b3 = jnp.concatenate([params['h%d_b3' % c] for c in range(4)])[None]

    k0p = jnp.zeros((1, FPAD), jnp.float32).at[0, :FDIM].set(
        params['cross_k0'][:, 0])
    b0p = jnp.zeros((1, FPAD), jnp.float32).at[0, :FDIM].set(
        params['cross_b0'][:, 0])
    k1p = jnp.zeros((1, FPAD), jnp.float32).at[0, :FDIM].set(
        params['cross_k1'][:, 0])
    b1p = jnp.zeros((1, FPAD), jnp.float32).at[0, :FDIM].set(
        params['cross_b1'][:, 0])

    mesh = plsc.VectorSubcoreMesh(core_axis_name="c", subcore_axis_name="s",
                                  num_cores=NC, num_subcores=NS)
    tu, tf, ta = _relayout_call(mesh=mesh)(
        params['userid_table'].T, params['feedid_table'].T,
        params['authorid_table'].T)
    merged = _gather_call(mesh=mesh)(
        tu.reshape(VPAD, E), tf.reshape(VPAD, E), ta.reshape(VPAD, E),
        params['device_table'], sid, mid, dense_pad)

    return _dense_call(merged, k0p, b0p, k1p, b1p,
                       w1cat, c1cat, w2s, c2s, w3s, b3)


# speedup vs baseline: 2.3552x; 2.3552x over previous
import functools

import jax
import jax.numpy as jnp
from jax import lax
from jax.experimental import pallas as pl
from jax.experimental.pallas import tpu as pltpu
from jax.experimental.pallas import tpu_sc as plsc

B = 4096
L = 50
E = 64
NUM_MULTI = 3
N_DENSE = 8
FDIM = 840
FPAD = 896
EPS = 1e-5

NC = 2
NS = 16
NW = NC * NS
BPW = B // NW
CB = 8
NCHUNK = BPW // CB
NSUB = 5
SUBI = CB * L // NSUB
IDXROWS = BPW * L // SUBI


def _seq_feature(mtab, sid_row, mid_h, feat_i, piece_h, sidx_v, midx_v,
                 srow_v, rows_v, out_v, sem_s, sem_g0, sem_g1, wid, base):
    gsems = (sem_g0, sem_g1)

    pltpu.sync_copy(sid_row.at[pl.ds(base, BPW)], sidx_v)
    pltpu.async_copy(mtab.at[sidx_v], srow_v, sem_s).wait()
    pltpu.sync_copy(srow_v, piece_h.at[pl.ds(base, BPW), pl.ds(0, E)])

    pltpu.sync_copy(mid_h.at[feat_i, wid], midx_v)

    def fire(c, k):
        for p in range(NSUB):
            pltpu.async_copy(
                mtab.at[midx_v.at[c * NSUB + p]],
                rows_v.at[k, pl.ds(p * SUBI, SUBI)],
                gsems[k])

    def drain(k):
        pltpu.make_async_copy(mtab.at[pl.ds(0, CB * L)], rows_v.at[k],
                              gsems[k]).wait()

    def compute_chunk(k):
        rows_k = rows_v.at[k]

        def b_body(b, _):
            r0 = b * L

            def l_body(l, acc):
                r = r0 + l
                a = list(acc)
                for e in range(4):
                    v = rows_k[r, pl.ds(e * 16, 16)]
                    a[e] = jnp.maximum(a[e], v)
                    a[4 + e] = jnp.minimum(a[4 + e], v)
                return tuple(a)

            v0s = tuple(rows_k[r0, pl.ds(e * 16, 16)] for e in range(4))
            acc = lax.fori_loop(1, L, l_body, v0s + v0s, unroll=7)
            for e in range(4):
                sl = pl.ds(e * 16, 16)
                mx = acc[e]
                mn = acc[4 + e]
                s = srow_v[b, sl]
                out_v[k, b, sl] = mx
                out_v[k, b, pl.ds(E + e * 16, 16)] = s - mn
                out_v[k, b, pl.ds(2 * E + e * 16, 16)] = jnp.where(
                    s >= 0.0, s * mx, s * mn)
            return 0

        lax.fori_loop(0, CB, b_body, 0)

    fire(0, 0)
    fire(1, 1)

    def pair_body(j, _):
        for k in range(2):
            c = j * 2 + k
            drain(k)
            compute_chunk(k)

            @pl.when(c + 2 < NCHUNK)
            def _f():
                fire(c + 2, k)

            pltpu.sync_copy(
                out_v.at[k],
                piece_h.at[pl.ds(base + c * CB, CB), pl.ds(E, 3 * E)])
        return 0

    lax.fori_loop(0, NCHUNK // 2, pair_body, 0)


def _sc_feat_main(mtab, t_dev, sid_h, mid_h, dense_h, piece_h, sidx_v,
                  midx_v, srow_v, rows_v, out_v, dtmp_v, sem_s, sem_g0,
                  sem_g1):
    wid = lax.axis_index("s") * NC + lax.axis_index("c")
    base = wid * BPW

    pltpu.sync_copy(sid_h.at[3, pl.ds(base, BPW)], sidx_v)
    pltpu.async_copy(t_dev.at[sidx_v], srow_v, sem_s).wait()
    pltpu.sync_copy(srow_v, piece_h.at[pl.ds(base, BPW), pl.ds(4 * E, E)])

    pltpu.sync_copy(dense_h.at[pl.ds(base, BPW)], dtmp_v)
    pltpu.sync_copy(dtmp_v, piece_h.at[pl.ds(base, BPW), pl.ds(5 * E, E)])

    _seq_feature(mtab, sid_h.at[1], mid_h, 0, piece_h, sidx_v, midx_v,
                 srow_v, rows_v, out_v, sem_s, sem_g0, sem_g1, wid, base)


def _sc_feat_auth(mtab, sid_h, mid_h, piece_h, sidx_v, midx_v, srow_v,
                  rows_v, out_v, sem_s, sem_g0, sem_g1):
    wid = lax.axis_index("s") * NC + lax.axis_index("c")
    base = wid * BPW
    _seq_feature(mtab, sid_h.at[2], mid_h, 1, piece_h, sidx_v, midx_v,
                 srow_v, rows_v, out_v, sem_s, sem_g0, sem_g1, wid, base)


def _sc_feat_user(mtab, sid_h, mid_h, piece_h, sidx_v, midx_v, srow_v,
                  rows_v, out_v, sem_s, sem_g0, sem_g1):
    wid = lax.axis_index("s") * NC + lax.axis_index("c")
    base = wid * BPW
    _seq_feature(mtab, sid_h.at[0], mid_h, 2, piece_h, sidx_v, midx_v,
                 srow_v, rows_v, out_v, sem_s, sem_g0, sem_g1, wid, base)


_SCRATCH_COMMON = [
    pltpu.VMEM((BPW,), jnp.int32),
    pltpu.VMEM((IDXROWS, SUBI), jnp.int32),
    pltpu.VMEM((BPW, E), jnp.float32),
    pltpu.VMEM((2, CB * L, E), jnp.float32),
    pltpu.VMEM((2, CB, 3 * E), jnp.float32),
]
_SEMS = [pltpu.SemaphoreType.DMA] * 3

_feat_main_call = functools.partial(
    pl.kernel,
    _sc_feat_main,
    out_type=jax.ShapeDtypeStruct((B, 6 * E), jnp.float32),
    scratch_types=_SCRATCH_COMMON + [pltpu.VMEM((BPW, E), jnp.float32)]
    + _SEMS,
    compiler_params=pltpu.CompilerParams(use_tc_tiling_on_sc=False),
)

_feat_auth_call = functools.partial(
    pl.kernel,
    _sc_feat_auth,
    out_type=jax.ShapeDtypeStruct((B, 4 * E), jnp.float32),
    scratch_types=_SCRATCH_COMMON + _SEMS,
    compiler_params=pltpu.CompilerParams(use_tc_tiling_on_sc=False),
)

_feat_user_call = functools.partial(
    pl.kernel,
    _sc_feat_user,
    out_type=jax.ShapeDtypeStruct((B, 4 * E), jnp.float32),
    scratch_types=_SCRATCH_COMMON + _SEMS,
    compiler_params=pltpu.CompilerParams(use_tc_tiling_on_sc=False),
)


def _dense_body(pf_ref, pa_ref, pu_ref, k0_ref, b0_ref, k1_ref, b1_ref,
                w1_ref, c1_ref, w2_ref, c2_ref, w3_ref, b3_ref, out_ref):
    x = jnp.concatenate([pf_ref[...], pa_ref[...], pu_ref[...]], axis=1)
    xw = jnp.sum(x * k0_ref[...], axis=1, keepdims=True)
    x1 = x * xw + b0_ref[...] + x
    xw1 = jnp.sum(x1 * k1_ref[...], axis=1, keepdims=True)
    x2 = x * xw1 + b1_ref[...] + x1
    h1 = jnp.dot(x2, w1_ref[...], preferred_element_type=jnp.float32)
    h1 = jnp.maximum(h1 + c1_ref[...], 0.0)
    outs = []
    for c in range(4):
        hc = h1[:, 256 * c:256 * (c + 1)]
        h2 = jnp.dot(hc, w2_ref[c], preferred_element_type=jnp.float32)
        h2 = jnp.maximum(h2 + c2_ref[c], 0.0)
        outs.append(jnp.sum(h2 * w3_ref[c], axis=1, keepdims=True))
    o = jnp.concatenate(outs, axis=1) + b3_ref[...]
    out_ref[...] = jax.nn.sigmoid(o)


TB = 512


def _dense_call(pf, pa, pu, k0p, b0p, k1p, b1p, w1, c1, w2, c2, w3, b3):
    return pl.pallas_call(
        _dense_body,
        grid=(B // TB,),
        in_specs=[
            pl.BlockSpec((TB, 6 * E), lambda i: (i, 0)),
            pl.BlockSpec((TB, 4 * E), lambda i: (i, 0)),
            pl.BlockSpec((TB, 4 * E), lambda i: (i, 0)),
            pl.BlockSpec((1, FPAD), lambda i: (0, 0)),
            pl.BlockSpec((1, FPAD), lambda i: (0, 0)),
            pl.BlockSpec((1, FPAD), lambda i: (0, 0)),
            pl.BlockSpec((1, FPAD), lambda i: (0, 0)),
            pl.BlockSpec((FPAD, 1024), lambda i: (0, 0)),
            pl.BlockSpec((1, 1024), lambda i: (0, 0)),
            pl.BlockSpec((4, 256, 128), lambda i: (0, 0, 0)),
            pl.BlockSpec((4, 1, 128), lambda i: (0, 0, 0)),
            pl.BlockSpec((4, 1, 128), lambda i: (0, 0, 0)),
            pl.BlockSpec((1, 4), lambda i: (0, 0)),
        ],
        out_specs=pl.BlockSpec((TB, 4), lambda i: (i, 0)),
        out_shape=jax.ShapeDtypeStruct((B, 4), jnp.float32),
    )(pf, pa, pu, k0p, b0p, k1p, b1p, w1, c1, w2, c2, w3, b3)


def _expand_rows(v):
    z = jnp.zeros((56,) + v.shape[1:], v.dtype)
    return jnp.concatenate([
        v[64:128],
        v[256:448],
        v[192:256],
        v[832:840],
        z[:56],
        v[128:192],
        v[448:640],
        v[0:64],
        v[640:832],
    ], axis=0)


def kernel(dense_features, single_id_concat, multi_id_concat, mask_concat,
           params):
    del mask_concat

    sid = single_id_concat[:, 0, :].T.astype(jnp.int32)
    mid = (multi_id_concat.astype(jnp.int32)
           .transpose(2, 0, 1)
           .reshape(NUM_MULTI, NW, IDXROWS, SUBI))
    dense_pad = jnp.concatenate(
        [dense_features.astype(jnp.float32),
         jnp.zeros((B, E - N_DENSE), jnp.float32)], axis=1)

    inv = (1.0 + EPS) ** -0.5
    w1l, c1l, w2l, c2l, w3l = [], [], [], [], []
    for c in range(4):
        g1 = params['h%d_g1' % c] * inv
        w1 = _expand_rows(params['h%d_W1' % c]) * g1[None, :]
        c1 = params['h%d_b1' % c] * g1 + params['h%d_be1' % c]
        w1l.append(jnp.zeros((FPAD, 256), jnp.float32)
                   .at[:, :200].set(w1))
        c1l.append(jnp.zeros((256,), jnp.float32).at[:200].set(c1))
        g2 = params['h%d_g2' % c] * inv
        w2 = params['h%d_W2' % c] * g2[None, :]
        c2 = params['h%d_b2' % c] * g2 + params['h%d_be2' % c]
        w2l.append(jnp.zeros((256, 128), jnp.float32)
                   .at[:200, :80].set(w2))
        c2l.append(jnp.zeros((128,), jnp.float32).at[:80].set(c2))
        w3l.append(jnp.zeros((128,), jnp.float32)
                   .at[:80].set(params['h%d_W3' % c][:, 0]))
    w1cat = jnp.concatenate(w1l, axis=1)
    c1cat = jnp.concatenate(c1l)[None]
    w2s = jnp.stack(w2l)
    c2s = jnp.stack(c2l)[:, None, :]
    w3s = jnp.stack(w3l)[:, None, :]
    b3 = jnp.concatenate([params['h%d_b3' % c] for c in range(4)])[None]

    k0p = _expand_rows(params['cross_k0'][:, 0])[None]
    b0p = _expand_rows(params['cross_b0'][:, 0])[None]
    k1p = _expand_rows(params['cross_k1'][:, 0])[None]
    b1p = _expand_rows(params['cross_b1'][:, 0])[None]

    mesh = plsc.VectorSubcoreMesh(core_axis_name="c", subcore_axis_name="s",
                                  num_cores=NC, num_subcores=NS)
    pf = _feat_main_call(mesh=mesh)(
        params['feedid_table'], params['device_table'], sid, mid, dense_pad)
    pa = _feat_auth_call(mesh=mesh)(params['authorid_table'], sid, mid)
    pu = _feat_user_call(mesh=mesh)(params['userid_table'], sid, mid)

    return _dense_call(pf, pa, pu, k0p, b0p, k1p, b1p,
                       w1cat, c1cat, w2s, c2s, w3s, b3)


# speedup vs baseline: 2.3702x; 1.0064x over previous
import functools

import jax
import jax.numpy as jnp
from jax import lax
from jax.experimental import pallas as pl
from jax.experimental.pallas import tpu as pltpu
from jax.experimental.pallas import tpu_sc as plsc

B = 4096
L = 50
E = 64
NUM_MULTI = 3
N_DENSE = 8
FDIM = 840
FPAD = 896
EPS = 1e-5

NC = 2
NS = 16
NW = NC * NS
BPW = B // NW
CB = 8
NCHUNK = BPW // CB
NSUB = 5
SUBI = CB * L // NSUB
IDXROWS = BPW * L // SUBI


def _seq_feature(mtab, sid_row, mid_h, feat_i, piece_h, sidx_v, midx_v,
                 srow_v, rows_v, out_v, sem_s, sem_g0, sem_g1, wid, base):
    gsems = (sem_g0, sem_g1)

    pltpu.sync_copy(sid_row.at[pl.ds(base, BPW)], sidx_v)
    pltpu.async_copy(mtab.at[sidx_v], srow_v, sem_s).wait()
    pltpu.sync_copy(srow_v, piece_h.at[pl.ds(base, BPW), pl.ds(0, E)])

    pltpu.sync_copy(mid_h.at[feat_i, wid], midx_v)

    def fire(c, k):
        for p in range(NSUB):
            pltpu.async_copy(
                mtab.at[midx_v.at[c * NSUB + p]],
                rows_v.at[k, pl.ds(p * SUBI, SUBI)],
                gsems[k])

    def drain(k):
        pltpu.make_async_copy(mtab.at[pl.ds(0, CB * L)], rows_v.at[k],
                              gsems[k]).wait()

    def compute_chunk(k):
        rows_k = rows_v.at[k]

        def b_body(b, _):
            r0 = b * L

            def l_body(l, acc):
                r = r0 + l
                a = list(acc)
                for e in range(4):
                    v = rows_k[r, pl.ds(e * 16, 16)]
                    a[e] = jnp.maximum(a[e], v)
                    a[4 + e] = jnp.minimum(a[4 + e], v)
                return tuple(a)

            v0s = tuple(rows_k[r0, pl.ds(e * 16, 16)] for e in range(4))
            acc = lax.fori_loop(1, L, l_body, v0s + v0s, unroll=7)
            for e in range(4):
                sl = pl.ds(e * 16, 16)
                mx = acc[e]
                mn = acc[4 + e]
                s = srow_v[b, sl]
                out_v[k, b, sl] = mx
                out_v[k, b, pl.ds(E + e * 16, 16)] = s - mn
                out_v[k, b, pl.ds(2 * E + e * 16, 16)] = jnp.where(
                    s >= 0.0, s * mx, s * mn)
            return 0

        lax.fori_loop(0, CB, b_body, 0)

    fire(0, 0)
    fire(1, 1)

    def pair_body(j, _):
        for k in range(2):
            c = j * 2 + k
            drain(k)
            compute_chunk(k)

            @pl.when(c + 2 < NCHUNK)
            def _f():
                fire(c + 2, k)

            pltpu.sync_copy(
                out_v.at[k],
                piece_h.at[pl.ds(base + c * CB, CB), pl.ds(E, 3 * E)])
        return 0

    lax.fori_loop(0, NCHUNK // 2, pair_body, 0)


def _sc_feat_main(mtab, t_dev, sid_h, mid_h, dense_h, piece_h, sidx_v,
                  midx_v, srow_v, rows_v, out_v, dtmp_v, sem_s, sem_g0,
                  sem_g1):
    wid = lax.axis_index("s") * NC + lax.axis_index("c")
    base = wid * BPW

    pltpu.sync_copy(sid_h.at[3, pl.ds(base, BPW)], sidx_v)
    pltpu.async_copy(t_dev.at[sidx_v], srow_v, sem_s).wait()
    pltpu.sync_copy(srow_v, piece_h.at[pl.ds(base, BPW), pl.ds(4 * E, E)])

    pltpu.sync_copy(dense_h.at[pl.ds(base, BPW)], dtmp_v)
    pltpu.sync_copy(dtmp_v, piece_h.at[pl.ds(base, BPW), pl.ds(5 * E, E)])

    _seq_feature(mtab, sid_h.at[1], mid_h, 0, piece_h, sidx_v, midx_v,
                 srow_v, rows_v, out_v, sem_s, sem_g0, sem_g1, wid, base)


def _sc_feat_auth(mtab, sid_h, mid_h, piece_h, sidx_v, midx_v, srow_v,
                  rows_v, out_v, sem_s, sem_g0, sem_g1):
    wid = lax.axis_index("s") * NC + lax.axis_index("c")
    base = wid * BPW
    _seq_feature(mtab, sid_h.at[2], mid_h, 1, piece_h, sidx_v, midx_v,
                 srow_v, rows_v, out_v, sem_s, sem_g0, sem_g1, wid, base)


def _sc_feat_user(mtab, sid_h, mid_h, piece_h, sidx_v, midx_v, srow_v,
                  rows_v, out_v, sem_s, sem_g0, sem_g1):
    wid = lax.axis_index("s") * NC + lax.axis_index("c")
    base = wid * BPW
    _seq_feature(mtab, sid_h.at[0], mid_h, 2, piece_h, sidx_v, midx_v,
                 srow_v, rows_v, out_v, sem_s, sem_g0, sem_g1, wid, base)


_SCRATCH_COMMON = [
    pltpu.VMEM((BPW,), jnp.int32),
    pltpu.VMEM((IDXROWS, SUBI), jnp.int32),
    pltpu.VMEM((BPW, E), jnp.float32),
    pltpu.VMEM((2, CB * L, E), jnp.float32),
    pltpu.VMEM((2, CB, 3 * E), jnp.float32),
]
_SEMS = [pltpu.SemaphoreType.DMA] * 3

_feat_main_call = functools.partial(
    pl.kernel,
    _sc_feat_main,
    out_type=jax.ShapeDtypeStruct((B, 6 * E), jnp.float32),
    scratch_types=_SCRATCH_COMMON + [pltpu.VMEM((BPW, E), jnp.float32)]
    + _SEMS,
    compiler_params=pltpu.CompilerParams(use_tc_tiling_on_sc=False),
)

_feat_auth_call = functools.partial(
    pl.kernel,
    _sc_feat_auth,
    out_type=jax.ShapeDtypeStruct((B, 4 * E), jnp.float32),
    scratch_types=_SCRATCH_COMMON + _SEMS,
    compiler_params=pltpu.CompilerParams(use_tc_tiling_on_sc=False),
)

_feat_user_call = functools.partial(
    pl.kernel,
    _sc_feat_user,
    out_type=jax.ShapeDtypeStruct((B, 4 * E), jnp.float32),
    scratch_types=_SCRATCH_COMMON + _SEMS,
    compiler_params=pltpu.CompilerParams(use_tc_tiling_on_sc=False),
)


def _dense_body(pf_ref, pa_ref, pu_ref, k0_ref, b0_ref, k1_ref, b1_ref,
                w1_ref, c1_ref, w2_ref, c2_ref, w3_ref, b3_ref, out_ref):
    x = jnp.concatenate([pf_ref[...], pa_ref[...], pu_ref[...]], axis=1)
    xw = jnp.sum(x * k0_ref[...], axis=1, keepdims=True)
    x1 = x * xw + b0_ref[...] + x
    xw1 = jnp.sum(x1 * k1_ref[...], axis=1, keepdims=True)
    x2 = x * xw1 + b1_ref[...] + x1
    h1 = jnp.dot(x2.astype(jnp.bfloat16), w1_ref[...],
                 preferred_element_type=jnp.float32)
    h1 = jnp.maximum(h1 + c1_ref[...], 0.0)
    outs = []
    for c in range(4):
        hc = h1[:, 256 * c:256 * (c + 1)]
        h2 = jnp.dot(hc.astype(jnp.bfloat16), w2_ref[c],
                     preferred_element_type=jnp.float32)
        h2 = jnp.maximum(h2 + c2_ref[c], 0.0)
        outs.append(jnp.sum(h2 * w3_ref[c], axis=1, keepdims=True))
    o = jnp.concatenate(outs, axis=1) + b3_ref[...]
    out_ref[...] = jax.nn.sigmoid(o)


TB = 512


def _dense_call(pf, pa, pu, k0p, b0p, k1p, b1p, w1, c1, w2, c2, w3, b3):
    return pl.pallas_call(
        _dense_body,
        grid=(B // TB,),
        in_specs=[
            pl.BlockSpec((TB, 6 * E), lambda i: (i, 0)),
            pl.BlockSpec((TB, 4 * E), lambda i: (i, 0)),
            pl.BlockSpec((TB, 4 * E), lambda i: (i, 0)),
            pl.BlockSpec((1, FPAD), lambda i: (0, 0)),
            pl.BlockSpec((1, FPAD), lambda i: (0, 0)),
            pl.BlockSpec((1, FPAD), lambda i: (0, 0)),
            pl.BlockSpec((1, FPAD), lambda i: (0, 0)),
            pl.BlockSpec((FPAD, 1024), lambda i: (0, 0)),
            pl.BlockSpec((1, 1024), lambda i: (0, 0)),
            pl.BlockSpec((4, 256, 128), lambda i: (0, 0, 0)),
            pl.BlockSpec((4, 1, 128), lambda i: (0, 0, 0)),
            pl.BlockSpec((4, 1, 128), lambda i: (0, 0, 0)),
            pl.BlockSpec((1, 4), lambda i: (0, 0)),
        ],
        out_specs=pl.BlockSpec((TB, 4), lambda i: (i, 0)),
        out_shape=jax.ShapeDtypeStruct((B, 4), jnp.float32),
    )(pf, pa, pu, k0p, b0p, k1p, b1p, w1, c1, w2, c2, w3, b3)


def _expand_rows(v):
    z = jnp.zeros((56,) + v.shape[1:], v.dtype)
    return jnp.concatenate([
        v[64:128],
        v[256:448],
        v[192:256],
        v[832:840],
        z[:56],
        v[128:192],
        v[448:640],
        v[0:64],
        v[640:832],
    ], axis=0)


def kernel(dense_features, single_id_concat, multi_id_concat, mask_concat,
           params):
    del mask_concat

    sid = single_id_concat[:, 0, :].T.astype(jnp.int32)
    mid = (multi_id_concat.astype(jnp.int32)
           .transpose(2, 0, 1)
           .reshape(NUM_MULTI, NW, IDXROWS, SUBI))
    dense_pad = jnp.concatenate(
        [dense_features.astype(jnp.float32),
         jnp.zeros((B, E - N_DENSE), jnp.float32)], axis=1)

    inv = (1.0 + EPS) ** -0.5
    w1l, c1l, w2l, c2l, w3l = [], [], [], [], []
    for c in range(4):
        g1 = params['h%d_g1' % c] * inv
        w1 = _expand_rows(params['h%d_W1' % c]) * g1[None, :]
        c1 = params['h%d_b1' % c] * g1 + params['h%d_be1' % c]
        w1l.append(jnp.zeros((FPAD, 256), jnp.float32)
                   .at[:, :200].set(w1))
        c1l.append(jnp.zeros((256,), jnp.float32).at[:200].set(c1))
        g2 = params['h%d_g2' % c] * inv
        w2 = params['h%d_W2' % c] * g2[None, :]
        c2 = params['h%d_b2' % c] * g2 + params['h%d_be2' % c]
        w2l.append(jnp.zeros((256, 128), jnp.float32)
                   .at[:200, :80].set(w2))
        c2l.append(jnp.zeros((128,), jnp.float32).at[:80].set(c2))
        w3l.append(jnp.zeros((128,), jnp.float32)
                   .at[:80].set(params['h%d_W3' % c][:, 0]))
    w1cat = jnp.concatenate(w1l, axis=1)
    c1cat = jnp.concatenate(c1l)[None]
    w2s = jnp.stack(w2l)
    c2s = jnp.stack(c2l)[:, None, :]
    w3s = jnp.stack(w3l)[:, None, :]
    b3 = jnp.concatenate([params['h%d_b3' % c] for c in range(4)])[None]

    k0p = _expand_rows(params['cross_k0'][:, 0])[None]
    b0p = _expand_rows(params['cross_b0'][:, 0])[None]
    k1p = _expand_rows(params['cross_k1'][:, 0])[None]
    b1p = _expand_rows(params['cross_b1'][:, 0])[None]

    mesh = plsc.VectorSubcoreMesh(core_axis_name="c", subcore_axis_name="s",
                                  num_cores=NC, num_subcores=NS)
    pf = _feat_main_call(mesh=mesh)(
        params['feedid_table'], params['device_table'], sid, mid, dense_pad)
    pa = _feat_auth_call(mesh=mesh)(params['authorid_table'], sid, mid)
    pu = _feat_user_call(mesh=mesh)(params['userid_table'], sid, mid)

    return _dense_call(pf, pa, pu, k0p, b0p, k1p, b1p,
                       w1cat.astype(jnp.bfloat16), c1cat,
                       w2s.astype(jnp.bfloat16), c2s, w3s, b3)


# speedup vs baseline: 2.4002x; 1.0127x over previous
import functools

import jax
import jax.numpy as jnp
from jax import lax
from jax.experimental import pallas as pl
from jax.experimental.pallas import tpu as pltpu
from jax.experimental.pallas import tpu_sc as plsc

B = 4096
L = 50
E = 64
NUM_MULTI = 3
N_DENSE = 8
FDIM = 840
FPAD = 896
EPS = 1e-5

NC = 2
NS = 16
NW = NC * NS
BPW = B // NW
CB = 8
NCHUNK = BPW // CB
NSUB = 5
SUBI = CB * L // NSUB
IDXROWS = BPW * L // SUBI


def _seq_feature(mtab, sid_row, mid_h, feat_i, piece_h, sidx_v, midl_v,
                 midx_v, srow_v, rows_v, out_v, sem_s, sem_g0, sem_g1,
                 wid, base):
    gsems = (sem_g0, sem_g1)

    pltpu.sync_copy(sid_row.at[pl.ds(base, BPW)], sidx_v)
    pltpu.async_copy(mtab.at[sidx_v], srow_v, sem_s).wait()
    pltpu.sync_copy(srow_v, piece_h.at[pl.ds(base, BPW), pl.ds(0, E)])

    pltpu.sync_copy(mid_h.at[feat_i, :, pl.ds(base, BPW)], midl_v)
    iota50 = lax.iota(jnp.int32, 16) * L

    def reorder(l, _):
        for bg in range(BPW // 16):
            v = midl_v[l, pl.ds(16 * bg, 16)]
            plsc.store_scatter(midx_v, [iota50 + (16 * bg * L + l)], v)
        return 0

    lax.fori_loop(0, L, reorder, 0)

    def fire(c, k):
        for p in range(NSUB):
            pltpu.async_copy(
                mtab.at[midx_v.at[pl.ds((c * NSUB + p) * SUBI, SUBI)]],
                rows_v.at[k, pl.ds(p * SUBI, SUBI)],
                gsems[k])

    def drain(k):
        pltpu.make_async_copy(mtab.at[pl.ds(0, CB * L)], rows_v.at[k],
                              gsems[k]).wait()

    def compute_chunk(k):
        rows_k = rows_v.at[k]

        def b_body(b, _):
            r0 = b * L

            def l_body(l, acc):
                r = r0 + l
                a = list(acc)
                for e in range(4):
                    v = rows_k[r, pl.ds(e * 16, 16)]
                    a[e] = jnp.maximum(a[e], v)
                    a[4 + e] = jnp.minimum(a[4 + e], v)
                return tuple(a)

            v0s = tuple(rows_k[r0, pl.ds(e * 16, 16)] for e in range(4))
            acc = lax.fori_loop(1, L, l_body, v0s + v0s, unroll=7)
            for e in range(4):
                sl = pl.ds(e * 16, 16)
                mx = acc[e]
                mn = acc[4 + e]
                s = srow_v[b, sl]
                out_v[k, b, sl] = mx
                out_v[k, b, pl.ds(E + e * 16, 16)] = s - mn
                out_v[k, b, pl.ds(2 * E + e * 16, 16)] = jnp.where(
                    s >= 0.0, s * mx, s * mn)
            return 0

        lax.fori_loop(0, CB, b_body, 0)

    fire(0, 0)
    fire(1, 1)

    def pair_body(j, _):
        for k in range(2):
            c = j * 2 + k
            drain(k)
            compute_chunk(k)

            @pl.when(c + 2 < NCHUNK)
            def _f():
                fire(c + 2, k)

            pltpu.sync_copy(
                out_v.at[k],
                piece_h.at[pl.ds(base + c * CB, CB), pl.ds(E, 3 * E)])
        return 0

    lax.fori_loop(0, NCHUNK // 2, pair_body, 0)


def _sc_feat_main(mtab, t_dev, sid_h, mid_h, dense_h, piece_h, sidx_v,
                  midl_v, midx_v, srow_v, rows_v, out_v, dtmp_v, sem_s,
                  sem_g0, sem_g1):
    wid = lax.axis_index("s") * NC + lax.axis_index("c")
    base = wid * BPW

    pltpu.sync_copy(sid_h.at[3, pl.ds(base, BPW)], sidx_v)
    pltpu.async_copy(t_dev.at[sidx_v], srow_v, sem_s).wait()
    pltpu.sync_copy(srow_v, piece_h.at[pl.ds(base, BPW), pl.ds(4 * E, E)])

    pltpu.sync_copy(dense_h.at[pl.ds(base, BPW)], dtmp_v)
    pltpu.sync_copy(dtmp_v, piece_h.at[pl.ds(base, BPW), pl.ds(5 * E, E)])

    _seq_feature(mtab, sid_h.at[1], mid_h, 0, piece_h, sidx_v, midl_v,
                 midx_v, srow_v, rows_v, out_v, sem_s, sem_g0, sem_g1,
                 wid, base)


def _sc_feat_auth(mtab, sid_h, mid_h, piece_h, sidx_v, midl_v, midx_v,
                  srow_v, rows_v, out_v, sem_s, sem_g0, sem_g1):
    wid = lax.axis_index("s") * NC + lax.axis_index("c")
    base = wid * BPW
    _seq_feature(mtab, sid_h.at[2], mid_h, 1, piece_h, sidx_v, midl_v,
                 midx_v, srow_v, rows_v, out_v, sem_s, sem_g0, sem_g1,
                 wid, base)


def _sc_feat_user(mtab, sid_h, mid_h, piece_h, sidx_v, midl_v, midx_v,
                  srow_v, rows_v, out_v, sem_s, sem_g0, sem_g1):
    wid = lax.axis_index("s") * NC + lax.axis_index("c")
    base = wid * BPW
    _seq_feature(mtab, sid_h.at[0], mid_h, 2, piece_h, sidx_v, midl_v,
                 midx_v, srow_v, rows_v, out_v, sem_s, sem_g0, sem_g1,
                 wid, base)


_SCRATCH_COMMON = [
    pltpu.VMEM((BPW,), jnp.int32),
    pltpu.VMEM((L, BPW), jnp.int32),
    pltpu.VMEM((BPW * L,), jnp.int32),
    pltpu.VMEM((BPW, E), jnp.float32),
    pltpu.VMEM((2, CB * L, E), jnp.float32),
    pltpu.VMEM((2, CB, 3 * E), jnp.float32),
]
_SEMS = [pltpu.SemaphoreType.DMA] * 3

_feat_main_call = functools.partial(
    pl.kernel,
    _sc_feat_main,
    out_type=jax.ShapeDtypeStruct((B, 6 * E), jnp.float32),
    scratch_types=_SCRATCH_COMMON + [pltpu.VMEM((BPW, E), jnp.float32)]
    + _SEMS,
    compiler_params=pltpu.CompilerParams(use_tc_tiling_on_sc=False,
                                         needs_layout_passes=False),
)

_feat_auth_call = functools.partial(
    pl.kernel,
    _sc_feat_auth,
    out_type=jax.ShapeDtypeStruct((B, 4 * E), jnp.float32),
    scratch_types=_SCRATCH_COMMON + _SEMS,
    compiler_params=pltpu.CompilerParams(use_tc_tiling_on_sc=False,
                                         needs_layout_passes=False),
)

_feat_user_call = functools.partial(
    pl.kernel,
    _sc_feat_user,
    out_type=jax.ShapeDtypeStruct((B, 4 * E), jnp.float32),
    scratch_types=_SCRATCH_COMMON + _SEMS,
    compiler_params=pltpu.CompilerParams(use_tc_tiling_on_sc=False,
                                         needs_layout_passes=False),
)


def _dense_body(pf_ref, pa_ref, pu_ref, k0_ref, b0_ref, k1_ref, b1_ref,
                w1_ref, c1_ref, w2_ref, c2_ref, w3_ref, b3_ref, out_ref):
    x = jnp.concatenate([pf_ref[...], pa_ref[...], pu_ref[...]], axis=1)
    xw = jnp.sum(x * k0_ref[...], axis=1, keepdims=True)
    x1 = x * xw + b0_ref[...] + x
    xw1 = jnp.sum(x1 * k1_ref[...], axis=1, keepdims=True)
    x2 = x * xw1 + b1_ref[...] + x1
    h1 = jnp.dot(x2.astype(jnp.bfloat16), w1_ref[...],
                 preferred_element_type=jnp.float32)
    h1 = jnp.maximum(h1 + c1_ref[...], 0.0)
    outs = []
    for c in range(4):
        hc = h1[:, 256 * c:256 * (c + 1)]
        h2 = jnp.dot(hc.astype(jnp.bfloat16), w2_ref[c],
                     preferred_element_type=jnp.float32)
        h2 = jnp.maximum(h2 + c2_ref[c], 0.0)
        outs.append(jnp.sum(h2 * w3_ref[c], axis=1, keepdims=True))
    o = jnp.concatenate(outs, axis=1) + b3_ref[...]
    out_ref[...] = jax.nn.sigmoid(o)


TB = 512


def _dense_call(pf, pa, pu, k0p, b0p, k1p, b1p, w1, c1, w2, c2, w3, b3):
    return pl.pallas_call(
        _dense_body,
        grid=(B // TB,),
        in_specs=[
            pl.BlockSpec((TB, 6 * E), lambda i: (i, 0)),
            pl.BlockSpec((TB, 4 * E), lambda i: (i, 0)),
            pl.BlockSpec((TB, 4 * E), lambda i: (i, 0)),
            pl.BlockSpec((1, FPAD), lambda i: (0, 0)),
            pl.BlockSpec((1, FPAD), lambda i: (0, 0)),
            pl.BlockSpec((1, FPAD), lambda i: (0, 0)),
            pl.BlockSpec((1, FPAD), lambda i: (0, 0)),
            pl.BlockSpec((FPAD, 1024), lambda i: (0, 0)),
            pl.BlockSpec((1, 1024), lambda i: (0, 0)),
            pl.BlockSpec((4, 256, 128), lambda i: (0, 0, 0)),
            pl.BlockSpec((4, 1, 128), lambda i: (0, 0, 0)),
            pl.BlockSpec((4, 1, 128), lambda i: (0, 0, 0)),
            pl.BlockSpec((1, 4), lambda i: (0, 0)),
        ],
        out_specs=pl.BlockSpec((TB, 4), lambda i: (i, 0)),
        out_shape=jax.ShapeDtypeStruct((B, 4), jnp.float32),
    )(pf, pa, pu, k0p, b0p, k1p, b1p, w1, c1, w2, c2, w3, b3)


def _expand_rows(v):
    z = jnp.zeros((56,) + v.shape[1:], v.dtype)
    return jnp.concatenate([
        v[64:128],
        v[256:448],
        v[192:256],
        v[832:840],
        z[:56],
        v[128:192],
        v[448:640],
        v[0:64],
        v[640:832],
    ], axis=0)


def kernel(dense_features, single_id_concat, multi_id_concat, mask_concat,
           params):
    del mask_concat

    sid = single_id_concat[:, 0, :].T.astype(jnp.int32)
    mid = multi_id_concat.astype(jnp.int32).transpose(2, 1, 0)
    dense_pad = jnp.concatenate(
        [dense_features.astype(jnp.float32),
         jnp.zeros((B, E - N_DENSE), jnp.float32)], axis=1)

    inv = (1.0 + EPS) ** -0.5
    w1l, c1l, w2l, c2l, w3l = [], [], [], [], []
    for c in range(4):
        g1 = params['h%d_g1' % c] * inv
        w1 = _expand_rows(params['h%d_W1' % c]) * g1[None, :]
        c1 = params['h%d_b1' % c] * g1 + params['h%d_be1' % c]
        w1l.append(jnp.zeros((FPAD, 256), jnp.float32)
                   .at[:, :200].set(w1))
        c1l.append(jnp.zeros((256,), jnp.float32).at[:200].set(c1))
        g2 = params['h%d_g2' % c] * inv
        w2 = params['h%d_W2' % c] * g2[None, :]
        c2 = params['h%d_b2' % c] * g2 + params['h%d_be2' % c]
        w2l.append(jnp.zeros((256, 128), jnp.float32)
                   .at[:200, :80].set(w2))
        c2l.append(jnp.zeros((128,), jnp.float32).at[:80].set(c2))
        w3l.append(jnp.zeros((128,), jnp.float32)
                   .at[:80].set(params['h%d_W3' % c][:, 0]))
    w1cat = jnp.concatenate(w1l, axis=1)
    c1cat = jnp.concatenate(c1l)[None]
    w2s = jnp.stack(w2l)
    c2s = jnp.stack(c2l)[:, None, :]
    w3s = jnp.stack(w3l)[:, None, :]
    b3 = jnp.concatenate([params['h%d_b3' % c] for c in range(4)])[None]

    k0p = _expand_rows(params['cross_k0'][:, 0])[None]
    b0p = _expand_rows(params['cross_b0'][:, 0])[None]
    k1p = _expand_rows(params['cross_k1'][:, 0])[None]
    b1p = _expand_rows(params['cross_b1'][:, 0])[None]

    mesh = plsc.VectorSubcoreMesh(core_axis_name="c", subcore_axis_name="s",
                                  num_cores=NC, num_subcores=NS)
    pf = _feat_main_call(mesh=mesh)(
        params['feedid_table'], params['device_table'], sid, mid, dense_pad)
    pa = _feat_auth_call(mesh=mesh)(params['authorid_table'], sid, mid)
    pu = _feat_user_call(mesh=mesh)(params['userid_table'], sid, mid)

    return _dense_call(pf, pa, pu, k0p, b0p, k1p, b1p,
                       w1cat.astype(jnp.bfloat16), c1cat,
                       w2s.astype(jnp.bfloat16), c2s, w3s, b3)
